# Initial kernel scaffold; baseline (speedup 1.0000x reference)
import jax
import jax.numpy as jnp
from jax.experimental import pallas as pl


def kernel(x):
    raise NotImplementedError("write your pallas kernel here")


# speedup vs baseline: 70.0953x; 70.0953x over previous
import jax
import jax.numpy as jnp
from jax import lax
from jax.experimental import pallas as pl
from jax.experimental.pallas import tpu as pltpu

_NEG = float("-inf")

_SORT8_NET = (
    (0, 1), (2, 3), (4, 5), (6, 7),
    (0, 2), (1, 3), (4, 6), (5, 7),
    (1, 2), (5, 6),
    (0, 4), (1, 5), (2, 6), (3, 7),
    (2, 4), (3, 5),
    (1, 2), (3, 4), (5, 6),
)

_CLEAN8_NET = (
    (0, 4), (1, 5), (2, 6), (3, 7),
    (0, 2), (1, 3), (4, 6), (5, 7),
    (0, 1), (2, 3), (4, 5), (6, 7),
)


def _cmpex(v, i, j):
    hi = jnp.maximum(v[i], v[j])
    lo = jnp.minimum(v[i], v[j])
    v[i] = hi
    v[j] = lo


def _sort8_desc(v):
    v = list(v)
    for i, j in _SORT8_NET:
        _cmpex(v, i, j)
    return v


def _merge_top8(r, s):
    m = [jnp.maximum(r[i], s[7 - i]) for i in range(8)]
    for i, j in _CLEAN8_NET:
        _cmpex(m, i, j)
    return m


def _topk_body(x_ref, o_ref):
    def body(g, r):
        r = list(r)
        base = g * 64
        v = [x_ref[0, pl.ds(base + 8 * i, 8), :] for i in range(8)]
        v = _sort8_desc(v)
        return tuple(_merge_top8(r, v))

    init = tuple(jnp.full((8, 128), _NEG, jnp.float32) for _ in range(8))
    r = lax.fori_loop(0, 64, body, init)

    c = jnp.concatenate(list(r), axis=0)
    rows = lax.broadcasted_iota(jnp.int32, (64, 128), 0)
    outs = []
    for _ in range(8):
        m = jnp.max(c, axis=0, keepdims=True)
        outs.append(m)
        occ = c == m
        idx = jnp.where(occ, rows, 64)
        imin = jnp.min(idx, axis=0, keepdims=True)
        c = jnp.where(occ & (rows == imin), _NEG, c)
    o_ref[0] = jnp.concatenate(outs, axis=0)


def kernel(x):
    out = pl.pallas_call(
        _topk_body,
        grid=(4, 8),
        in_specs=[pl.BlockSpec((1, 4096, 128), lambda b, cb: (b, 0, cb))],
        out_specs=pl.BlockSpec((1, 8, 128), lambda b, cb: (b, 0, cb)),
        out_shape=jax.ShapeDtypeStruct((4, 8, 1024), jnp.float32),
        compiler_params=pltpu.CompilerParams(
            dimension_semantics=("parallel", "parallel"),
        ),
    )(x)
    return jnp.transpose(out, (0, 2, 1)).reshape(4, 8 * 1024)
